# baseline (device time: 33443 ns/iter reference)
import functools

import jax
import jax.numpy as jnp
from jax import lax
from jax.experimental import pallas as pl
from jax.experimental.pallas import tpu as pltpu

N_CHUNKS = 1


def kernel(x):
    m, n = x.shape
    half = m // 2
    r = half // N_CHUNKS

    def body(x_ref, out_ref, ysend_sems, yrecv_sems, zsend_sems, zrecv_sems):
        my_x = lax.axis_index("x")
        my_y = lax.axis_index("y")
        my_z = lax.axis_index("z")
        nbr_y = (my_x, 1 - my_y, my_z)
        nbr_z = (my_x, my_y, 1 - my_z)

        barrier_sem = pltpu.get_barrier_semaphore()
        for nbr in (nbr_y, nbr_z):
            pl.semaphore_signal(
                barrier_sem, inc=1, device_id=nbr,
                device_id_type=pl.DeviceIdType.MESH,
            )
        pl.semaphore_wait(barrier_sem, 2)

        send_base = my_z * half
        y_dst_base = my_y * m + my_z * half
        y_recv_base = (1 - my_y) * m + my_z * half
        z_recv_base = (1 - my_y) * m + (1 - my_z) * half

        y_rdmas = []
        for c in range(N_CHUNKS):
            rd = pltpu.make_async_remote_copy(
                src_ref=x_ref.at[pl.ds(send_base + c * r, r), :],
                dst_ref=out_ref.at[pl.ds(y_dst_base + c * r, r), :],
                send_sem=ysend_sems.at[c],
                recv_sem=yrecv_sems.at[c],
                device_id=nbr_y,
                device_id_type=pl.DeviceIdType.MESH,
            )
            rd.start()
            y_rdmas.append(rd)

        out_ref[pl.ds(my_y * m, m), :] = x_ref[...]

        z_rdmas = []
        for c in range(N_CHUNKS):
            y_rdmas[c].wait_recv()
            rd = pltpu.make_async_remote_copy(
                src_ref=out_ref.at[pl.ds(y_recv_base + c * r, r), :],
                dst_ref=out_ref.at[pl.ds(y_recv_base + c * r, r), :],
                send_sem=zsend_sems.at[c],
                recv_sem=zrecv_sems.at[c],
                device_id=nbr_z,
                device_id_type=pl.DeviceIdType.MESH,
            )
            rd.start()
            z_rdmas.append(rd)

        for c in range(N_CHUNKS):
            z_rdmas[c].wait_recv()
        for c in range(N_CHUNKS):
            y_rdmas[c].wait_send()
            z_rdmas[c].wait_send()

        @functools.partial(
            pl.run_scoped, second_barrier=pltpu.SemaphoreType.REGULAR
        )
        def _(second_barrier):
            for nbr in (nbr_y, nbr_z):
                pl.semaphore_signal(
                    second_barrier, inc=1, device_id=nbr,
                    device_id_type=pl.DeviceIdType.MESH,
                )
            pl.semaphore_wait(second_barrier, 2)

    return pl.pallas_call(
        body,
        out_shape=jax.ShapeDtypeStruct((2 * m, n), x.dtype),
        in_specs=[pl.BlockSpec(memory_space=pltpu.VMEM)],
        out_specs=pl.BlockSpec(memory_space=pltpu.VMEM),
        scratch_shapes=[
            pltpu.SemaphoreType.DMA((N_CHUNKS,)),
            pltpu.SemaphoreType.DMA((N_CHUNKS,)),
            pltpu.SemaphoreType.DMA((N_CHUNKS,)),
            pltpu.SemaphoreType.DMA((N_CHUNKS,)),
        ],
        compiler_params=pltpu.CompilerParams(collective_id=0),
    )(x)


# device time: 20616 ns/iter; 1.6222x vs baseline; 1.6222x over previous
import os

import jax
import jax.numpy as jnp
from jax import lax
from jax.experimental import pallas as pl
from jax.experimental.pallas import tpu as pltpu

QC = int(os.environ.get("AG_QCHUNKS", "8"))


def kernel(x):
    m, n = x.shape
    q = m // 4
    r = q // QC
    d_y = 88
    d_x = 88
    d_z = 80

    def body(x_ref, out_ref, ysend, yrecv, xsend, xrecv, zsend, zrecv,
             own_sem):
        my_x = lax.axis_index("x")
        my_y = lax.axis_index("y")
        my_z = lax.axis_index("z")
        nbr_y = (my_x, 1 - my_y, my_z)
        nbr_x = (1 - my_x, my_y, my_z)
        nbr_z = (my_x, my_y, 1 - my_z)

        barrier_sem = pltpu.get_barrier_semaphore()
        for nbr in (nbr_y, nbr_x, nbr_z):
            pl.semaphore_signal(
                barrier_sem, inc=1, device_id=nbr,
                device_id_type=pl.DeviceIdType.MESH,
            )
        pl.semaphore_wait(barrier_sem, 3)

        own = my_y * m
        mis = (1 - my_y) * m
        q_me = (2 * my_x + my_z) * q
        q_x = (2 * (1 - my_x) + my_z) * q
        q_z = (2 * my_x + (1 - my_z)) * q
        q_d = (2 * (1 - my_x) + (1 - my_z)) * q

        def rdma(src, dst, ssem, rsem, dev):
            return pltpu.make_async_remote_copy(
                src_ref=src, dst_ref=dst, send_sem=ssem, recv_sem=rsem,
                device_id=dev, device_id_type=pl.DeviceIdType.MESH,
            )

        y_rdmas = []
        for c in range(QC):
            rd = rdma(
                x_ref.at[pl.ds(q_me + c * r, r), :],
                out_ref.at[pl.ds(own + q_me + c * r, r), :],
                ysend.at[c], yrecv.at[c], nbr_y,
            )
            rd.start()
            y_rdmas.append(rd)
        y_diag = rdma(
            x_ref.at[pl.ds(q_d, d_y), :],
            out_ref.at[pl.ds(own + q_d, d_y), :],
            ysend.at[QC], yrecv.at[QC], nbr_y,
        )
        y_diag.start()

        own_cp = pltpu.make_async_copy(
            x_ref, out_ref.at[pl.ds(own, m), :], own_sem
        )
        own_cp.start()

        x_rdmas = []
        z_rdmas = []
        for c in range(QC):
            y_rdmas[c].wait_recv()
            src = out_ref.at[pl.ds(mis + q_me + c * r, r), :]
            rdx = rdma(src, src, xsend.at[c], xrecv.at[c], nbr_x)
            rdx.start()
            x_rdmas.append(rdx)
            rdz = rdma(src, src, zsend.at[c], zrecv.at[c], nbr_z)
            rdz.start()
            z_rdmas.append(rdz)

        x_waited = set()
        z_waited = set()
        for c in range(d_y // r, (d_y + d_x - 1) // r + 1):
            z_rdmas[c].wait_recv()
            z_waited.add(c)
        x2 = rdma(
            out_ref.at[pl.ds(mis + q_z + d_y, d_x), :],
            out_ref.at[pl.ds(mis + q_z + d_y, d_x), :],
            xsend.at[QC], xrecv.at[QC], nbr_x,
        )
        x2.start()
        for c in range((d_y + d_x) // r, (q - 1) // r + 1):
            x_rdmas[c].wait_recv()
            x_waited.add(c)
        z2 = rdma(
            out_ref.at[pl.ds(mis + q_x + d_y + d_x, d_z), :],
            out_ref.at[pl.ds(mis + q_x + d_y + d_x, d_z), :],
            zsend.at[QC], zrecv.at[QC], nbr_z,
        )
        z2.start()

        y_diag.wait_recv()
        for c in range(QC):
            if c not in x_waited:
                x_rdmas[c].wait_recv()
            if c not in z_waited:
                z_rdmas[c].wait_recv()
        x2.wait_recv()
        z2.wait_recv()
        own_cp.wait()
        for rd in y_rdmas + x_rdmas + z_rdmas + [y_diag, x2, z2]:
            rd.wait_send()


    return pl.pallas_call(
        body,
        out_shape=jax.ShapeDtypeStruct((2 * m, n), x.dtype),
        in_specs=[pl.BlockSpec(memory_space=pltpu.VMEM)],
        out_specs=pl.BlockSpec(memory_space=pltpu.VMEM),
        scratch_shapes=[
            pltpu.SemaphoreType.DMA((QC + 1,)),
            pltpu.SemaphoreType.DMA((QC + 1,)),
            pltpu.SemaphoreType.DMA((QC + 1,)),
            pltpu.SemaphoreType.DMA((QC + 1,)),
            pltpu.SemaphoreType.DMA((QC + 1,)),
            pltpu.SemaphoreType.DMA((QC + 1,)),
            pltpu.SemaphoreType.DMA,
        ],
        compiler_params=pltpu.CompilerParams(collective_id=0),
    )(x)
